# baseline (speedup 1.0000x reference)
import jax
import jax.numpy as jnp
from jax import lax
from jax.experimental import pallas as pl
from jax.experimental.pallas import tpu as pltpu
from jax.experimental.pallas import tpu_sc as plsc

N = 2048
C = 128
L = 16
NCH = N // L
NC, NS = 2, 16
NW = NC * NS
CPW = C // NW
UNROLL = 4
UNROLL_SHIFT = 2
BUNROLL = 2
EPS = 1e-10


def _splat_last(v):
    idx = jnp.full((L, 1), L - 1, jnp.int32)
    return lax.gather(
        v, idx,
        dimension_numbers=lax.GatherDimensionNumbers(
            offset_dims=(), collapsed_slice_dims=(0,), start_index_map=(0,)),
        slice_sizes=(1,),
        mode=lax.GatherScatterMode.PROMISE_IN_BOUNDS)


def _search_interp(G_ref, q, wc, t):
    res = jnp.zeros((L,), jnp.int32)
    step = N // 2
    while step >= 1:
        probe = plsc.load_gather(G_ref, [res + (step - 1)])
        res = jnp.where(probe < q, res + step, res)
        step //= 2
    idx = jnp.maximum(res - 1, 0)
    g0 = plsc.load_gather(G_ref, [idx])
    g1 = plsc.load_gather(G_ref, [idx + 1])
    frac = (q - g0) / (g1 - g0 + EPS)
    diff = t - (idx.astype(jnp.float32) + frac)
    return diff * diff * wc


def _wd_body(x_hbm, y_hbm, out_hbm, f_v, g_v, gp_v, gn_v,
             qp_v, qn_v, wp_v, wn_v, tp_v, tn_v, res_v):
    cid = lax.axis_index("c")
    sid = lax.axis_index("s")
    wid = sid * NC + cid
    iota_f = lax.iota(jnp.int32, L).astype(jnp.float32)
    zi = jnp.int32(0)
    zv = jnp.zeros((L,), jnp.float32)

    acc = jnp.zeros((L,), jnp.float32)
    for j in range(CPW):
        col = wid * CPW + j
        pltpu.sync_copy(x_hbm.at[col], f_v)
        pltpu.sync_copy(y_hbm.at[col], g_v)

        def sums_body(i, carry):
            wp, wn, gp, gn = carry
            fc = f_v[pl.ds(i * L, L)]
            gc = g_v[pl.ds(i * L, L)]
            return (wp + jnp.maximum(fc, 0.0), wn + jnp.maximum(-fc, 0.0),
                    gp + jnp.maximum(gc, 0.0), gn + jnp.maximum(-gc, 0.0))

        swp, swn, sgp, sgn = lax.fori_loop(
            0, NCH, sums_body, (zv, zv, zv, zv))
        one_v = 1.0 + zv
        rwp = one_v / (jnp.sum(swp) + zv)
        rwn = one_v / (jnp.sum(swn) + zv)
        rgp = one_v / (jnp.sum(sgp) + zv)
        rgn = one_v / (jnp.sum(sgn) + zv)

        def build_chunk(i, carry):
            cgp, cgn, cfp, cfn, np_, nn_ = carry
            gc = g_v[pl.ds(i * L, L)]
            fc = f_v[pl.ds(i * L, L)]
            gpc = jnp.maximum(gc, 0.0) * rgp
            gnc = jnp.maximum(-gc, 0.0) * rgn
            bp = plsc.cumsum(gpc)
            bn = plsc.cumsum(gnc)
            gp_v[pl.ds(i * L, L)] = bp + cgp
            gn_v[pl.ds(i * L, L)] = bn + cgn
            wcp = jnp.maximum(fc, 0.0) * rwp
            wcn = jnp.maximum(-fc, 0.0) * rwn
            rp = plsc.cumsum(wcp)
            rn = plsc.cumsum(wcn)
            qp = rp + cfp
            qn = rn + cfn
            t = (i * L).astype(jnp.float32) + iota_f
            mp = wcp > 0.0
            mn = wcn > 0.0
            plsc.store_compressed(qp_v.at[pl.ds(np_, L)], qp, mask=mp)
            plsc.store_compressed(wp_v.at[pl.ds(np_, L)], wcp, mask=mp)
            plsc.store_compressed(tp_v.at[pl.ds(np_, L)], t, mask=mp)
            plsc.store_compressed(qn_v.at[pl.ds(nn_, L)], qn, mask=mn)
            plsc.store_compressed(wn_v.at[pl.ds(nn_, L)], wcn, mask=mn)
            plsc.store_compressed(tn_v.at[pl.ds(nn_, L)], t, mask=mn)
            np2 = np_ + jnp.sum(mp.astype(jnp.int32))
            nn2 = nn_ + jnp.sum(mn.astype(jnp.int32))
            return (cgp + _splat_last(bp), cgn + _splat_last(bn),
                    cfp + _splat_last(rp), cfn + _splat_last(rn), np2, nn2)

        def build_body(i, carry):
            for u in range(BUNROLL):
                carry = build_chunk(i * BUNROLL + u, carry)
            return carry

        _, _, _, _, nnp, nnn = lax.fori_loop(
            0, NCH // BUNROLL, build_body, (zv, zv, zv, zv, zi, zi))

        wp_v[pl.ds(nnp, L)] = zv
        qp_v[pl.ds(nnp, L)] = zv
        tp_v[pl.ds(nnp, L)] = zv
        wn_v[pl.ds(nnn, L)] = zv
        qn_v[pl.ds(nnn, L)] = zv
        tn_v[pl.ds(nnn, L)] = zv

        ncp = lax.shift_right_arithmetic(nnp + (L - 1), 4)
        ncn = lax.shift_right_arithmetic(nnn + (L - 1), 4)
        ncm = jnp.maximum(ncp, ncn)
        niter = lax.shift_right_arithmetic(ncm + (UNROLL - 1), UNROLL_SHIFT)

        def side_terms(G_ref, q_ref, w_ref, t_ref, k, nc):
            kc = jnp.maximum(jnp.minimum(k, nc - 1), 0)
            off = kc * L
            q = q_ref[pl.ds(off, L)]
            wc = w_ref[pl.ds(off, L)]
            t = t_ref[pl.ds(off, L)]
            flag = jnp.where(k < nc, 1.0, 0.0)
            return _search_interp(G_ref, q, wc, t) * flag

        def query_body(i, a):
            for u in range(UNROLL):
                k = i * UNROLL + u
                a = a + side_terms(gp_v, qp_v, wp_v, tp_v, k, ncp)
                a = a + side_terms(gn_v, qn_v, wn_v, tn_v, k, ncn)
            return a

        acc = lax.fori_loop(0, niter, query_body, acc)

    res_v[...] = acc
    pltpu.sync_copy(res_v, out_hbm.at[wid])


_sc_call = pl.kernel(
    _wd_body,
    out_type=jax.ShapeDtypeStruct((NW, L), jnp.float32),
    mesh=plsc.VectorSubcoreMesh(core_axis_name="c", subcore_axis_name="s"),
    compiler_params=pltpu.CompilerParams(needs_layout_passes=False),
    scratch_types=[
        pltpu.VMEM((N,), jnp.float32),
        pltpu.VMEM((N,), jnp.float32),
        pltpu.VMEM((N,), jnp.float32),
        pltpu.VMEM((N,), jnp.float32),
        pltpu.VMEM((N + L,), jnp.float32),
        pltpu.VMEM((N + L,), jnp.float32),
        pltpu.VMEM((N + L,), jnp.float32),
        pltpu.VMEM((N + L,), jnp.float32),
        pltpu.VMEM((N + L,), jnp.float32),
        pltpu.VMEM((N + L,), jnp.float32),
        pltpu.VMEM((L,), jnp.float32),
    ],
)


def kernel(x, y):
    xT = x.reshape(N, C).T
    yT = y.reshape(N, C).T
    part = _sc_call(xT, yT)
    return jnp.sum(part)
